# baseline (speedup 1.0000x reference)
import functools

import jax
import jax.numpy as jnp
from jax import lax
from jax.experimental import pallas as pl
from jax.experimental.pallas import tpu as pltpu
from jax.experimental.pallas import tpu_sc as plsc

_RING_SLICES = ((0, 18), (22, 26), (30, 34), (38, 42), (46, 64))

_NW = 32


def _pool_kernel(img_ref, tok0_ref, out_ref):
    out_ref[0:64] = tok0_ref[...]
    r = lax.broadcasted_iota(jnp.int32, (512, 128), 0)
    c = lax.broadcasted_iota(jnp.int32, (512, 128), 1)
    s4 = (r // 4 == c).astype(jnp.float32)
    s2 = (r[:256] // 2 == c[:256]).astype(jnp.float32)

    s4b = s4.astype(jnp.bfloat16)
    s2b = s2.astype(jnp.bfloat16)

    def split3(x):
        hi = x.astype(jnp.bfloat16)
        r = x - hi.astype(jnp.float32)
        mid = r.astype(jnp.bfloat16)
        lo = (r - mid.astype(jnp.float32)).astype(jnp.bfloat16)
        return hi, mid, lo

    def poolrows(s, parts):
        prods = [lax.dot_general(s, p, ((((0,), (0,))), ((), ())),
                                 preferred_element_type=jnp.float32)
                 for p in parts]
        return (prods[0] + prods[1]) + prods[2]

    def poolcols(x, s):
        parts = [lax.dot_general(p, s, ((((1,), (0,))), ((), ())),
                                 preferred_element_type=jnp.float32)
                 for p in split3(x)]
        return (parts[0] + parts[1]) + parts[2]

    for ch in range(3):
        img_parts = split3(img_ref[ch])
        sub_parts = [p[128:384, 128:384] for p in img_parts]
        rows2 = poolrows(s2b, sub_parts)
        p1 = jnp.floor(poolcols(rows2, s2b) * 0.25)
        rows4 = poolrows(s4b, img_parts)
        p2 = jnp.floor(poolcols(rows4, s4b) * 0.0625)

        ring_pos = {}
        for p, k in enumerate(k for a, b in _RING_SLICES for k in range(a, b)):
            ring_pos[k] = p
        for base, plane in ((64, p1), (112, p2)):
            for x in range(8):
                tcol = plane[:, 16 * x:16 * x + 16].reshape(8, 16, 16)
                for y in range(8):
                    k = y * 8 + x
                    if k in ring_pos:
                        out_ref[base + ring_pos[k], ch] = tcol[y]


_TOK0_PER_W = 64 // _NW


@functools.cache
def _make_l0_gather():
    @functools.partial(
        pl.kernel,
        out_type=jax.ShapeDtypeStruct((160, 3, 16, 16), jnp.float32),
        scratch_types=[
            pltpu.VMEM((1, 3, 16, 128), jnp.float32),
            pltpu.VMEM((_TOK0_PER_W, 3, 16, 16), jnp.float32),
            pltpu.SemaphoreType.DMA,
            pltpu.SemaphoreType.DMA,
        ],
        mesh=plsc.VectorSubcoreMesh(core_axis_name="c", subcore_axis_name="s"),
    )
    def _l0_gather(img_hbm, out_hbm, stage_v, tok_v, gsem, ssem):
        wid = lax.axis_index("s") * 2 + lax.axis_index("c")
        n0 = 2 * wid
        y, x0 = n0 // 8, n0 % 8
        win = jnp.where(x0 < 4, 128, 256)
        pltpu.async_copy(
            img_hbm.at[:, pl.ds(pl.multiple_of(192 + 16 * y, 16), 16),
                       pl.ds(pl.multiple_of(win, 128), 128)],
            stage_v.at[0], gsem).wait()
        stores = []
        for t in range(_TOK0_PER_W):
            col = pl.multiple_of(192 + 16 * (x0 + t) - win, 16)
            for c in range(3):
                for a in range(16):
                    v = stage_v[0, c, a, pl.ds(col, 16)]
                    tok_v[t, c, a, :] = v.astype(jnp.int32).astype(jnp.float32)
            stores.append(pltpu.async_copy(tok_v.at[t], out_hbm.at[n0 + t], ssem))
        for cp in stores:
            cp.wait()

    return _l0_gather


def kernel(images):
    tok0 = _make_l0_gather()(images)
    return pl.pallas_call(
        _pool_kernel,
        out_shape=jax.ShapeDtypeStruct((160, 3, 16, 16), jnp.float32),
        grid=(1,),
        in_specs=[
            pl.BlockSpec((3, 512, 512), lambda i: (0, 0, 0)),
            pl.BlockSpec((64, 3, 16, 16), lambda i: (0, 0, 0, 0)),
        ],
        out_specs=pl.BlockSpec((160, 3, 16, 16), lambda i: (0, 0, 0, 0)),
        input_output_aliases={1: 0},
    )(images, tok0)


# speedup vs baseline: 1.2093x; 1.2093x over previous
import functools

import jax
import jax.numpy as jnp
from jax import lax
from jax.experimental import pallas as pl
from jax.experimental.pallas import tpu as pltpu
from jax.experimental.pallas import tpu_sc as plsc

_RING_SLICES = ((0, 18), (22, 26), (30, 34), (38, 42), (46, 64))

_NW = 32


def _pool_kernel(img_ref, out_ref):
    r = lax.broadcasted_iota(jnp.int32, (512, 128), 0)
    c = lax.broadcasted_iota(jnp.int32, (512, 128), 1)
    s4 = (r // 4 == c).astype(jnp.float32)
    s2 = (r[:256] // 2 == c[:256]).astype(jnp.float32)

    s4b = s4.astype(jnp.bfloat16)
    s2b = s2.astype(jnp.bfloat16)

    def split3(x):
        hi = x.astype(jnp.bfloat16)
        r = x - hi.astype(jnp.float32)
        mid = r.astype(jnp.bfloat16)
        lo = (r - mid.astype(jnp.float32)).astype(jnp.bfloat16)
        return hi, mid, lo

    def poolrows(s, parts):
        prods = [lax.dot_general(s, p, ((((0,), (0,))), ((), ())),
                                 preferred_element_type=jnp.float32)
                 for p in parts]
        return (prods[0] + prods[1]) + prods[2]

    def poolcols(x, s):
        parts = [lax.dot_general(p, s, ((((1,), (0,))), ((), ())),
                                 preferred_element_type=jnp.float32)
                 for p in split3(x)]
        return (parts[0] + parts[1]) + parts[2]

    for ch in range(3):
        img_parts = split3(img_ref[ch])
        sub_parts = [p[128:384, 128:384] for p in img_parts]
        rows2 = poolrows(s2b, sub_parts)
        p1 = jnp.floor(poolcols(rows2, s2b) * 0.25)
        rows4 = poolrows(s4b, img_parts)
        p2 = jnp.floor(poolcols(rows4, s4b) * 0.0625)

        ring_pos = {}
        for p, k in enumerate(k for a, b in _RING_SLICES for k in range(a, b)):
            ring_pos[k] = p
        for base, plane in ((0, p1), (48, p2)):
            for x in range(8):
                tcol = plane[:, 16 * x:16 * x + 16].reshape(8, 16, 16)
                for y in range(8):
                    k = y * 8 + x
                    if k in ring_pos:
                        out_ref[base + ring_pos[k], ch] = tcol[y]


_TOK0_PER_W = 64 // _NW


@functools.cache
def _make_l0_gather():
    @functools.partial(
        pl.kernel,
        out_type=jax.ShapeDtypeStruct((64, 3, 16, 16), jnp.float32),
        scratch_types=[
            pltpu.VMEM((1, 3, 16, 128), jnp.float32),
            pltpu.VMEM((_TOK0_PER_W, 3, 16, 16), jnp.float32),
            pltpu.SemaphoreType.DMA,
            pltpu.SemaphoreType.DMA,
        ],
        mesh=plsc.VectorSubcoreMesh(core_axis_name="c", subcore_axis_name="s"),
    )
    def _l0_gather(img_hbm, out_hbm, stage_v, tok_v, gsem, ssem):
        wid = lax.axis_index("s") * 2 + lax.axis_index("c")
        n0 = 2 * wid
        y, x0 = n0 // 8, n0 % 8
        win = jnp.where(x0 < 4, 128, 256)
        pltpu.async_copy(
            img_hbm.at[:, pl.ds(pl.multiple_of(192 + 16 * y, 16), 16),
                       pl.ds(pl.multiple_of(win, 128), 128)],
            stage_v.at[0], gsem).wait()
        stores = []
        for t in range(_TOK0_PER_W):
            col = pl.multiple_of(192 + 16 * (x0 + t) - win, 16)
            for c in range(3):
                for a in range(16):
                    v = stage_v[0, c, a, pl.ds(col, 16)]
                    tok_v[t, c, a, :] = v.astype(jnp.int32).astype(jnp.float32)
            stores.append(pltpu.async_copy(tok_v.at[t], out_hbm.at[n0 + t], ssem))
        for cp in stores:
            cp.wait()

    return _l0_gather


def kernel(images):
    tok0 = _make_l0_gather()(images)
    tok12 = pl.pallas_call(
        _pool_kernel,
        out_shape=jax.ShapeDtypeStruct((96, 3, 16, 16), jnp.float32),
    )(images)
    return jnp.concatenate([tok0, tok12], axis=0)
